# baseline (speedup 1.0000x reference)
import functools

import jax
import jax.numpy as jnp
from jax import lax
from jax.experimental import pallas as pl
from jax.experimental.pallas import tpu as pltpu
from jax.experimental.pallas import tpu_sc as plsc

_B, _C, _H, _W = 8, 96, 224, 224
_N = 2000

_NUM_WORKERS = 32
_NPAD = 2048
_PTS = _NPAD // _NUM_WORKERS
_LANES = 16
_GCHUNK = 8

_RROWS = 1024
_ROWS = _B * _C * _H
_PLANE = _ROWS * 128


_CPB = 8


def _repack_body(in_ref, a_ref, b_ref):
    x = in_ref[0]
    a_ref[...] = x[:, :, 0:128].reshape(a_ref.shape)
    b_ref[...] = x[:, :, 96:224].reshape(b_ref.shape)


def _repack(input):
    blk = _CPB * _H * 128
    nc = _C // _CPB
    return pl.pallas_call(
        _repack_body,
        grid=(_B * nc,),
        in_specs=[pl.BlockSpec((1, _CPB, _H, _W),
                               lambda i: (i // nc, i % nc, 0, 0))],
        out_specs=[pl.BlockSpec((blk,), lambda i: (i,)),
                   pl.BlockSpec((blk,), lambda i: (i,))],
        out_shape=[jax.ShapeDtypeStruct((_PLANE,), jnp.float32),
                   jax.ShapeDtypeStruct((_PLANE,), jnp.float32)],
    )(input)


def _sc_gather_kernel(pa, pb, ids_hbm, rows_hbm, cols_hbm, out_hbm,
                      ids_v, rows_v, cols_v, idxa_v, idxb_v,
                      ga_v, gb_v, g_v, sem):
    nc = lax.axis_size("c")
    wid = lax.axis_index("s") * nc + lax.axis_index("c")
    base = wid * _PTS

    pltpu.sync_copy(ids_hbm.at[pl.ds(base, _PTS)], ids_v)
    pltpu.sync_copy(rows_hbm.at[pl.ds(base, _PTS)], rows_v)
    pltpu.sync_copy(cols_hbm.at[pl.ds(base, _PTS)], cols_v)

    pbase_a, pbase_b, self_f = [], [], []
    for gr in range(_PTS // _LANES):
        sl = pl.ds(gr * _LANES, _LANES)
        col = cols_v[sl]
        rbase = ids_v[sl] * (_C * _H) + rows_v[sl]
        sel = lax.shift_right_logical(col, 7)
        pbase_a.append(rbase * 128 + jnp.minimum(col, 127))
        pbase_b.append(rbase * 128 + jnp.maximum(col, 96) - 96)
        self_f.append(sel.astype(jnp.float32))

    def build(c, carry):
        coff = c * (_H * 128)
        for gr in range(_PTS // _LANES):
            sl = pl.ds(c * _PTS + gr * _LANES, _LANES)
            idxa_v[sl] = pbase_a[gr] + coff
            idxb_v[sl] = pbase_b[gr] + coff
        return carry

    lax.fori_loop(0, _C, build, 0)

    def gather_chunk(i, carry):
        cb = i * _GCHUNK
        descs = []
        for j in range(_GCHUNK):
            c = cb + j
            descs.append(pltpu.async_copy(
                pa.at[idxa_v.at[pl.ds(c * _PTS, _PTS)]], ga_v.at[c], sem))
            descs.append(pltpu.async_copy(
                pb.at[idxb_v.at[pl.ds(c * _PTS, _PTS)]], gb_v.at[c], sem))
        for d in descs:
            d.wait()
        return carry

    lax.fori_loop(0, _C // _GCHUNK, gather_chunk, 0)

    def select(c, carry):
        for gr in range(_PTS // _LANES):
            sl = pl.ds(gr * _LANES, _LANES)
            s = self_f[gr]
            g_v[c, sl] = ga_v[c, sl] * (1.0 - s) + gb_v[c, sl] * s
        return carry

    lax.fori_loop(0, _C, select, 0)

    pltpu.sync_copy(g_v, out_hbm.at[wid])


@functools.cache
def _sc_gather():
    return pl.kernel(
        _sc_gather_kernel,
        out_type=jax.ShapeDtypeStruct((_NUM_WORKERS, _C, _PTS), jnp.float32),
        mesh=plsc.VectorSubcoreMesh(
            core_axis_name="c", subcore_axis_name="s",
            num_cores=2, num_subcores=16,
        ),
        scratch_types=[
            pltpu.VMEM((_PTS,), jnp.int32),
            pltpu.VMEM((_PTS,), jnp.int32),
            pltpu.VMEM((_PTS,), jnp.int32),
            pltpu.VMEM((_C * _PTS,), jnp.int32),
            pltpu.VMEM((_C * _PTS,), jnp.int32),
            pltpu.VMEM((_C, _PTS), jnp.float32),
            pltpu.VMEM((_C, _PTS), jnp.float32),
            pltpu.VMEM((_C, _PTS), jnp.float32),
            pltpu.SemaphoreType.DMA,
        ],
    )


_WCHUNK = 8


def _bcast_body(g_ref, out_ref):
    g = g_ref[...]
    m = jnp.concatenate([g[i] for i in range(_NUM_WORKERS)], axis=1)
    out_ref[...] = jnp.broadcast_to(m[:, None, :_N], out_ref.shape)


def _tc_broadcast(g):
    return pl.pallas_call(
        _bcast_body,
        grid=(_W // _WCHUNK,),
        in_specs=[pl.BlockSpec((_NUM_WORKERS, _C, _PTS), lambda i: (0, 0, 0))],
        out_specs=pl.BlockSpec((_C, _WCHUNK, _N), lambda i: (0, i, 0)),
        out_shape=jax.ShapeDtypeStruct((_C, _W, _N), jnp.float32),
    )(g)


def kernel(input, image_ids, centers):
    pad = _NPAD - _N
    ids = jnp.pad(image_ids.astype(jnp.int32), (0, pad))
    rows = jnp.pad(centers[:, 0].astype(jnp.int32), (0, pad))
    cols = jnp.pad(centers[:, 1].astype(jnp.int32), (0, pad))
    pa, pb = _repack(input)
    g = _sc_gather()(pa, pb, ids, rows, cols)
    return jnp.transpose(_tc_broadcast(g), (2, 0, 1))
